# Initial kernel scaffold; baseline (speedup 1.0000x reference)
import jax
import jax.numpy as jnp
from jax.experimental import pallas as pl


def kernel(z_e, embedding_weight):
    raise NotImplementedError("write your pallas kernel here")


# speedup vs baseline: 1.2421x; 1.2421x over previous
import functools

import jax
import jax.numpy as jnp
from jax import lax
from jax.experimental import pallas as pl
from jax.experimental.pallas import tpu as pltpu

NUM_EMBEDDINGS = 8192
EMBEDDING_DIM = 256
TOKEN_TILE = 256


def _argmin_body(z_ref, e_ref, z2_ref, e2_ref, out_ref):
    mm = lax.dot_general(
        z_ref[...], e_ref[...],
        (((1,), (1,)), ((), ())),
        preferred_element_type=jnp.float32,
    )
    d = (z2_ref[...] + e2_ref[...]) - 2.0 * mm
    m = jnp.min(d, axis=1, keepdims=True)
    iota = lax.broadcasted_iota(jnp.int32, d.shape, 1)
    idx = jnp.min(jnp.where(d == m, iota, jnp.int32(NUM_EMBEDDINGS)), axis=1)
    out_ref[...] = idx


@functools.partial(jax.jit, static_argnames=())
def _encode(z_flat, embedding_weight, z2, e2):
    n_tok = z_flat.shape[0]
    grid = (n_tok // TOKEN_TILE,)
    return pl.pallas_call(
        _argmin_body,
        grid=grid,
        in_specs=[
            pl.BlockSpec((TOKEN_TILE, EMBEDDING_DIM), lambda i: (i, 0)),
            pl.BlockSpec((NUM_EMBEDDINGS, EMBEDDING_DIM), lambda i: (0, 0)),
            pl.BlockSpec((TOKEN_TILE, 1), lambda i: (i, 0)),
            pl.BlockSpec((1, NUM_EMBEDDINGS), lambda i: (0, 0)),
        ],
        out_specs=pl.BlockSpec((TOKEN_TILE,), lambda i: (i,)),
        out_shape=jax.ShapeDtypeStruct((n_tok,), jnp.int32),
    )(z_flat, embedding_weight, z2, e2)


def kernel(z_e, embedding_weight):
    B, C, H, W = z_e.shape
    z_flat = jnp.transpose(z_e.reshape(B, C, H * W), (0, 2, 1))
    z2 = jnp.sum(z_flat ** 2, axis=2, keepdims=True)
    e2 = jnp.sum(embedding_weight ** 2, axis=1)
    idx = _encode(
        z_flat.reshape(B * H * W, C),
        embedding_weight,
        z2.reshape(B * H * W, 1),
        e2.reshape(1, NUM_EMBEDDINGS),
    )
    encoding_indices = idx.reshape(B, H * W)
    quantized = jnp.take(embedding_weight, encoding_indices, axis=0)
    quantized = jnp.transpose(quantized, (0, 2, 1)).reshape(B, C, H, W)
    return (quantized, encoding_indices)
